# baseline (speedup 1.0000x reference)
import functools

import jax
import jax.numpy as jnp
from jax import lax
from jax.experimental import pallas as pl
from jax.experimental.pallas import tpu as pltpu
from jax.experimental.pallas import tpu_sc as plsc

N = 10000
E = 160000
F_IN = 256
F_HID = 256
F_OUT = 40

NP_ = 10240
EP = 163840
NC = 2
NS = 16
L = 16
K = 64
EC = EP // NC
ES = EC // NS
NCHUNK = ES // K
ROWS = NP_ // NS
BM = 1024
F2P = 128


def _sc_scatter_sum(D, gather, n0=NCHUNK, n1=NCHUNK, ntables=1):
    mesh = plsc.VectorSubcoreMesh(core_axis_name="c", subcore_axis_name="s")
    CROWS = max(n0, n1)
    NBUF = 2
    assert n0 % NBUF == 0 and n1 % NBUF == 0

    scratch = []
    if gather:
        scratch.append(pltpu.VMEM((CROWS, K), jnp.int32))
    scratch += [pltpu.VMEM((K,), jnp.int32) for _ in range(NBUF)]
    if gather:
        scratch += [pltpu.VMEM((K, D), jnp.float32) for _ in range(NBUF)]
    else:
        scratch += [pltpu.VMEM((K, D), jnp.float32),
                    pltpu.VMEM((K, D), jnp.float32)]
    scratch.append(pltpu.VMEM_SHARED((NP_, D), jnp.float32))
    scratch += [pltpu.SemaphoreType.DMA for _ in range(NBUF)]
    if gather:
        scratch += [pltpu.SemaphoreType.DMA for _ in range(NBUF)]

    def body(*refs):
        if gather:
            tables = refs[:ntables]
            src_hbm, dst_hbm = refs[ntables:ntables + 2]
            outs = refs[ntables + 2:2 * ntables + 2]
            rest = refs[2 * ntables + 2:]
            sidx, rest = rest[0], rest[1:]
        else:
            dst_hbm, out_hbm = refs[:2]
            tables, outs = (None,), (out_hbm,)
            rest = refs[2:]
        didx, rest = rest[:NBUF], rest[NBUF:]
        if gather:
            rows, rest = rest[:NBUF], rest[NBUF:]
            zbuf = rows[0]
        else:
            ones, zbuf = rest[0], rest[1]
            rest = rest[2:]
        acc, rest = rest[0], rest[1:]
        dsem, rest = rest[:NBUF], rest[NBUF:]
        gsem = rest[:NBUF] if gather else None

        c = lax.axis_index("c")
        s = lax.axis_index("s")
        row0 = jnp.where(c == 0, s * n0, NS * n0 + s * n1)
        nchunks = jnp.where(c == 0, n0, n1)

        if not gather:
            @pl.loop(0, K)
            def _(r):
                @pl.loop(0, D, step=L)
                def _(f):
                    ones.at[r, pl.ds(f, L)][...] = jnp.ones((L,), jnp.float32)

        if gather:
            @pl.when(nchunks > 0)
            def _():
                pltpu.sync_copy(src_hbm.at[pl.ds(row0, CROWS)], sidx)

        for table_hbm, out_hbm in zip(tables, outs):
            @pl.loop(0, K)
            def _(r):
                @pl.loop(0, D, step=L)
                def _(f):
                    zbuf.at[r, pl.ds(f, L)][...] = jnp.zeros((L,),
                                                             jnp.float32)

            @pl.loop(0, ROWS, step=K)
            def _(r0):
                pltpu.sync_copy(zbuf, acc.at[pl.ds(s * ROWS + r0, K)])

            @pl.when(nchunks > 0)
            def _():
                for b in range(NBUF):
                    @pl.when(nchunks > b)
                    def _(b=b):
                        pltpu.async_copy(dst_hbm.at[row0 + b], didx[b],
                                         dsem[b])
                        if gather:
                            pltpu.async_copy(table_hbm.at[sidx.at[b]],
                                             rows[b], gsem[b])

            plsc.subcore_barrier()

            def chunk_group(i, carry):
                j = i * NBUF
                for b in range(NBUF):
                    jj = j + b
                    pltpu.make_async_copy(dst_hbm.at[row0 + jj], didx[b],
                                          dsem[b]).wait()
                    if gather:
                        pltpu.make_async_copy(table_hbm.at[sidx.at[jj]],
                                              rows[b], gsem[b]).wait()
                        rbuf = rows[b]
                    else:
                        rbuf = ones
                    pltpu.sync_copy(rbuf, acc.at[didx[b]], add=True)

                    @pl.when(jj + NBUF < nchunks)
                    def _(b=b, jj=jj, rbuf=rbuf):
                        pltpu.async_copy(dst_hbm.at[row0 + jj + NBUF],
                                         didx[b], dsem[b])
                        if gather:
                            pltpu.async_copy(
                                table_hbm.at[sidx.at[jj + NBUF]], rbuf,
                                gsem[b])
                return carry

            lax.fori_loop(0, nchunks // NBUF, chunk_group, 0)

            plsc.subcore_barrier()
            pltpu.sync_copy(acc.at[pl.ds(s * ROWS, ROWS)],
                            out_hbm.at[pl.ds(c * NP_ + s * ROWS, ROWS)])

    out_t = jax.ShapeDtypeStruct((NC * NP_, D), jnp.float32)
    return functools.partial(
        pl.kernel,
        out_type=[out_t] * ntables if (gather and ntables > 1) else out_t,
        mesh=mesh,
        scratch_types=scratch,
    )(body)


_deg_kernel = None
_agg128_kernel = None
_agg2x_kernel = None


def _get_sc_kernels():
    global _deg_kernel, _agg128_kernel, _agg2x_kernel
    if _deg_kernel is None:
        _deg_kernel = _sc_scatter_sum(128, gather=False)
        _agg128_kernel = _sc_scatter_sum(128, gather=True, n0=160, n1=0)
        _agg2x_kernel = _sc_scatter_sum(128, gather=True, n0=160, n1=0,
                                        ntables=2)
    return _deg_kernel, _agg128_kernel, _agg2x_kernel


def _mm1_body(x_ref, w_ref, o_ref):
    o_ref[...] = jnp.dot(x_ref[...], w_ref[...],
                         preferred_element_type=jnp.float32)


def _scale_body(h_ref, da_ref, db_ref, dis_ref, ha_ref, hb_ref):
    deg = da_ref[:, 0:1] + db_ref[:, 0:1] + 1.0
    dis = lax.rsqrt(deg)
    dis_ref[...] = dis
    hs = h_ref[...] * dis
    ha_ref[...] = hs[:, :128]
    hb_ref[...] = hs[:, 128:]


def _layer2_body(agga_ref, aggb_ref, ha_ref, hb_ref, dis_ref, b1_ref, w2_ref,
                 o_ref):
    agg_a = agga_ref[0] + agga_ref[1] + ha_ref[...]
    agg_b = aggb_ref[0] + aggb_ref[1] + hb_ref[...]
    dis = dis_ref[...]
    pre = jnp.concatenate([agg_a, agg_b], axis=1) * dis + b1_ref[...]
    out1 = jnp.maximum(pre, 0.0)
    h2 = jnp.dot(out1, w2_ref[...], preferred_element_type=jnp.float32)
    o_ref[...] = h2 * dis


def _out_body(agg_ref, h2s_ref, dis_ref, b2_ref, o_ref):
    tot = (agg_ref[0] + agg_ref[1] + h2s_ref[...]) * dis_ref[...] + b2_ref[...]
    col = lax.broadcasted_iota(jnp.int32, tot.shape, 1)
    valid = col < F_OUT
    masked = jnp.where(valid, tot, -jnp.inf)
    m = jnp.max(masked, axis=1, keepdims=True)
    ex = jnp.where(valid, jnp.exp(tot - m), 0.0)
    lse = jnp.log(jnp.sum(ex, axis=1, keepdims=True))
    o_ref[...] = tot - m - lse


def kernel(x, edge_index, W1, b1, W2, b2):
    deg_k, agg128_k, agg2x_k = _get_sc_kernels()

    pad_ids = jnp.full((EP - E,), N, dtype=jnp.int32)
    over = jnp.full((64, K), N, dtype=jnp.int32)
    srcp = jnp.concatenate(
        [jnp.concatenate([edge_index[0], pad_ids]).reshape(EP // K, K), over])
    dstp = jnp.concatenate(
        [jnp.concatenate([edge_index[1], pad_ids]).reshape(EP // K, K), over])
    x_pad = jnp.pad(x, ((0, NP_ - N), (0, 0)))
    W2p = jnp.pad(W2, ((0, 0), (0, F2P - F_OUT)))
    b1r = b1.reshape(1, F_HID)
    b2r = jnp.pad(b2, (0, F2P - F_OUT)).reshape(1, F2P)

    grid = (NP_ // BM,)

    deg2 = deg_k(dstp).reshape(NC, NP_, 128)

    h1 = pl.pallas_call(
        _mm1_body,
        grid=grid,
        in_specs=[
            pl.BlockSpec((BM, F_IN), lambda i: (i, 0)),
            pl.BlockSpec((F_IN, F_HID), lambda i: (0, 0)),
        ],
        out_specs=pl.BlockSpec((BM, F_HID), lambda i: (i, 0)),
        out_shape=jax.ShapeDtypeStruct((NP_, F_HID), jnp.float32),
    )(x_pad, W1)

    dis, h1sa, h1sb = pl.pallas_call(
        _scale_body,
        grid=grid,
        in_specs=[
            pl.BlockSpec((BM, F_HID), lambda i: (i, 0)),
            pl.BlockSpec((BM, 128), lambda i: (i, 0)),
            pl.BlockSpec((BM, 128), lambda i: (i, 0)),
        ],
        out_specs=[
            pl.BlockSpec((BM, 1), lambda i: (i, 0)),
            pl.BlockSpec((BM, 128), lambda i: (i, 0)),
            pl.BlockSpec((BM, 128), lambda i: (i, 0)),
        ],
        out_shape=[
            jax.ShapeDtypeStruct((NP_, 1), jnp.float32),
            jax.ShapeDtypeStruct((NP_, 128), jnp.float32),
            jax.ShapeDtypeStruct((NP_, 128), jnp.float32),
        ],
    )(h1, deg2[0], deg2[1])

    agga, aggb = agg2x_k(h1sa, h1sb, srcp, dstp)
    agga = agga.reshape(NC, NP_, 128)
    aggb = aggb.reshape(NC, NP_, 128)

    h2s = pl.pallas_call(
        _layer2_body,
        grid=grid,
        in_specs=[
            pl.BlockSpec((NC, BM, 128), lambda i: (0, i, 0)),
            pl.BlockSpec((NC, BM, 128), lambda i: (0, i, 0)),
            pl.BlockSpec((BM, 128), lambda i: (i, 0)),
            pl.BlockSpec((BM, 128), lambda i: (i, 0)),
            pl.BlockSpec((BM, 1), lambda i: (i, 0)),
            pl.BlockSpec((1, F_HID), lambda i: (0, 0)),
            pl.BlockSpec((F_HID, F2P), lambda i: (0, 0)),
        ],
        out_specs=pl.BlockSpec((BM, F2P), lambda i: (i, 0)),
        out_shape=jax.ShapeDtypeStruct((NP_, F2P), jnp.float32),
    )(agga, aggb, h1sa, h1sb, dis, b1r, W2p)

    agg2 = agg128_k(h2s, srcp, dstp).reshape(NC, NP_, F2P)

    out = pl.pallas_call(
        _out_body,
        grid=grid,
        in_specs=[
            pl.BlockSpec((NC, BM, F2P), lambda i: (0, i, 0)),
            pl.BlockSpec((BM, F2P), lambda i: (i, 0)),
            pl.BlockSpec((BM, 1), lambda i: (i, 0)),
            pl.BlockSpec((1, F2P), lambda i: (0, 0)),
        ],
        out_specs=pl.BlockSpec((BM, F2P), lambda i: (i, 0)),
        out_shape=jax.ShapeDtypeStruct((NP_, F2P), jnp.float32),
    )(agg2, h2s, dis, b2r)

    return out[:N, :F_OUT]


# speedup vs baseline: 1.3114x; 1.3114x over previous
import functools

import jax
import jax.numpy as jnp
from jax import lax
from jax.experimental import pallas as pl
from jax.experimental.pallas import tpu as pltpu
from jax.experimental.pallas import tpu_sc as plsc

N = 10000
E = 160000
F_IN = 256
F_HID = 256
F_OUT = 40

NP_ = 10240
EP = 163840
NC = 2
NS = 16
L = 16
K = 128
EC = EP // NC
ES = EC // NS
NCHUNK = ES // K
ROWS = NP_ // NS
BM = 1024
F2P = 128


def _sc_scatter_sum(D, gather, n0=NCHUNK, n1=NCHUNK, ntables=1):
    mesh = plsc.VectorSubcoreMesh(core_axis_name="c", subcore_axis_name="s")
    CROWS = max(n0, n1)
    NBUF = 2
    assert n0 % NBUF == 0 and n1 % NBUF == 0

    scratch = []
    if gather:
        scratch.append(pltpu.VMEM((CROWS, K), jnp.int32))
    scratch += [pltpu.VMEM((K,), jnp.int32) for _ in range(NBUF)]
    if gather:
        scratch += [pltpu.VMEM((K, D), jnp.float32) for _ in range(NBUF)]
    else:
        scratch += [pltpu.VMEM((K, D), jnp.float32),
                    pltpu.VMEM((K, D), jnp.float32)]
    scratch.append(pltpu.VMEM_SHARED((NP_, D), jnp.float32))
    scratch += [pltpu.SemaphoreType.DMA for _ in range(NBUF)]
    if gather:
        scratch += [pltpu.SemaphoreType.DMA for _ in range(NBUF)]

    def body(*refs):
        if gather:
            tables = refs[:ntables]
            src_hbm, dst_hbm = refs[ntables:ntables + 2]
            outs = refs[ntables + 2:2 * ntables + 2]
            rest = refs[2 * ntables + 2:]
            sidx, rest = rest[0], rest[1:]
        else:
            dst_hbm, out_hbm = refs[:2]
            tables, outs = (None,), (out_hbm,)
            rest = refs[2:]
        didx, rest = rest[:NBUF], rest[NBUF:]
        if gather:
            rows, rest = rest[:NBUF], rest[NBUF:]
            zbuf = rows[0]
        else:
            ones, zbuf = rest[0], rest[1]
            rest = rest[2:]
        acc, rest = rest[0], rest[1:]
        dsem, rest = rest[:NBUF], rest[NBUF:]
        gsem = rest[:NBUF] if gather else None

        c = lax.axis_index("c")
        s = lax.axis_index("s")
        row0 = jnp.where(c == 0, s * n0, NS * n0 + s * n1)
        nchunks = jnp.where(c == 0, n0, n1)

        if not gather:
            @pl.loop(0, K)
            def _(r):
                @pl.loop(0, D, step=L)
                def _(f):
                    ones.at[r, pl.ds(f, L)][...] = jnp.ones((L,), jnp.float32)

        if gather:
            @pl.when(nchunks > 0)
            def _():
                pltpu.sync_copy(src_hbm.at[pl.ds(row0, CROWS)], sidx)

        for table_hbm, out_hbm in zip(tables, outs):
            @pl.loop(0, K)
            def _(r):
                @pl.loop(0, D, step=L)
                def _(f):
                    zbuf.at[r, pl.ds(f, L)][...] = jnp.zeros((L,),
                                                             jnp.float32)

            @pl.loop(0, ROWS, step=K)
            def _(r0):
                pltpu.sync_copy(zbuf, acc.at[pl.ds(s * ROWS + r0, K)])

            @pl.when(nchunks > 0)
            def _():
                for b in range(NBUF):
                    @pl.when(nchunks > b)
                    def _(b=b):
                        pltpu.async_copy(dst_hbm.at[row0 + b], didx[b],
                                         dsem[b])
                        if gather:
                            pltpu.async_copy(table_hbm.at[sidx.at[b]],
                                             rows[b], gsem[b])

            plsc.subcore_barrier()

            def chunk_group(i, carry):
                j = i * NBUF
                for b in range(NBUF):
                    jj = j + b
                    pltpu.make_async_copy(dst_hbm.at[row0 + jj], didx[b],
                                          dsem[b]).wait()
                    if gather:
                        pltpu.make_async_copy(table_hbm.at[sidx.at[jj]],
                                              rows[b], gsem[b]).wait()
                        rbuf = rows[b]
                    else:
                        rbuf = ones
                    pltpu.sync_copy(rbuf, acc.at[didx[b]], add=True)

                    @pl.when(jj + NBUF < nchunks)
                    def _(b=b, jj=jj, rbuf=rbuf):
                        pltpu.async_copy(dst_hbm.at[row0 + jj + NBUF],
                                         didx[b], dsem[b])
                        if gather:
                            pltpu.async_copy(
                                table_hbm.at[sidx.at[jj + NBUF]], rbuf,
                                gsem[b])
                return carry

            lax.fori_loop(0, nchunks // NBUF, chunk_group, 0)

            plsc.subcore_barrier()
            pltpu.sync_copy(acc.at[pl.ds(s * ROWS, ROWS)],
                            out_hbm.at[pl.ds(c * NP_ + s * ROWS, ROWS)])

    out_t = jax.ShapeDtypeStruct((NC * NP_, D), jnp.float32)
    return functools.partial(
        pl.kernel,
        out_type=[out_t] * ntables if (gather and ntables > 1) else out_t,
        mesh=mesh,
        scratch_types=scratch,
    )(body)


_deg_kernel = None
_agg128_kernel = None
_agg2x_kernel = None


def _get_sc_kernels():
    global _deg_kernel, _agg128_kernel, _agg2x_kernel
    if _deg_kernel is None:
        _deg_kernel = _sc_scatter_sum(128, gather=False)
        _agg128_kernel = _sc_scatter_sum(128, gather=True, n0=72, n1=8)
        _agg2x_kernel = _sc_scatter_sum(128, gather=True, n0=72, n1=8,
                                        ntables=2)
    return _deg_kernel, _agg128_kernel, _agg2x_kernel


def _mm1_body(x_ref, w_ref, o_ref):
    o_ref[...] = jnp.dot(x_ref[...], w_ref[...],
                         preferred_element_type=jnp.float32)


def _scale_body(h_ref, da_ref, db_ref, dis_ref, ha_ref, hb_ref):
    deg = da_ref[:, 0:1] + db_ref[:, 0:1] + 1.0
    dis = lax.rsqrt(deg)
    dis_ref[...] = dis
    hs = h_ref[...] * dis
    ha_ref[...] = hs[:, :128]
    hb_ref[...] = hs[:, 128:]


def _layer2_body(agga_ref, aggb_ref, ha_ref, hb_ref, dis_ref, b1_ref, w2_ref,
                 o_ref):
    agg_a = agga_ref[0] + agga_ref[1] + ha_ref[...]
    agg_b = aggb_ref[0] + aggb_ref[1] + hb_ref[...]
    dis = dis_ref[...]
    pre = jnp.concatenate([agg_a, agg_b], axis=1) * dis + b1_ref[...]
    out1 = jnp.maximum(pre, 0.0)
    h2 = jnp.dot(out1, w2_ref[...], preferred_element_type=jnp.float32)
    o_ref[...] = h2 * dis


def _out_body(agg_ref, h2s_ref, dis_ref, b2_ref, o_ref):
    tot = (agg_ref[0] + agg_ref[1] + h2s_ref[...]) * dis_ref[...] + b2_ref[...]
    col = lax.broadcasted_iota(jnp.int32, tot.shape, 1)
    valid = col < F_OUT
    masked = jnp.where(valid, tot, -jnp.inf)
    m = jnp.max(masked, axis=1, keepdims=True)
    ex = jnp.where(valid, jnp.exp(tot - m), 0.0)
    lse = jnp.log(jnp.sum(ex, axis=1, keepdims=True))
    o_ref[...] = (tot - m - lse)[:, :F_OUT]


def kernel(x, edge_index, W1, b1, W2, b2):
    deg_k, agg128_k, agg2x_k = _get_sc_kernels()

    pad_ids = jnp.full((EP - E,), N, dtype=jnp.int32)
    over = jnp.full((64, K), N, dtype=jnp.int32)
    srcp = jnp.concatenate(
        [jnp.concatenate([edge_index[0], pad_ids]).reshape(EP // K, K), over])
    dstp = jnp.concatenate(
        [jnp.concatenate([edge_index[1], pad_ids]).reshape(EP // K, K), over])
    x_pad = jnp.pad(x, ((0, NP_ - N), (0, 0)))
    W2p = jnp.pad(W2, ((0, 0), (0, F2P - F_OUT)))
    b1r = b1.reshape(1, F_HID)
    b2r = jnp.pad(b2, (0, F2P - F_OUT)).reshape(1, F2P)

    grid = (NP_ // BM,)

    deg2 = deg_k(dstp).reshape(NC, NP_, 128)

    h1 = pl.pallas_call(
        _mm1_body,
        grid=grid,
        in_specs=[
            pl.BlockSpec((BM, F_IN), lambda i: (i, 0)),
            pl.BlockSpec((F_IN, F_HID), lambda i: (0, 0)),
        ],
        out_specs=pl.BlockSpec((BM, F_HID), lambda i: (i, 0)),
        out_shape=jax.ShapeDtypeStruct((NP_, F_HID), jnp.float32),
    )(x_pad, W1)

    dis, h1sa, h1sb = pl.pallas_call(
        _scale_body,
        grid=grid,
        in_specs=[
            pl.BlockSpec((BM, F_HID), lambda i: (i, 0)),
            pl.BlockSpec((BM, 128), lambda i: (i, 0)),
            pl.BlockSpec((BM, 128), lambda i: (i, 0)),
        ],
        out_specs=[
            pl.BlockSpec((BM, 1), lambda i: (i, 0)),
            pl.BlockSpec((BM, 128), lambda i: (i, 0)),
            pl.BlockSpec((BM, 128), lambda i: (i, 0)),
        ],
        out_shape=[
            jax.ShapeDtypeStruct((NP_, 1), jnp.float32),
            jax.ShapeDtypeStruct((NP_, 128), jnp.float32),
            jax.ShapeDtypeStruct((NP_, 128), jnp.float32),
        ],
    )(h1, deg2[0], deg2[1])

    agga, aggb = agg2x_k(h1sa, h1sb, srcp, dstp)
    agga = agga.reshape(NC, NP_, 128)
    aggb = aggb.reshape(NC, NP_, 128)

    h2s = pl.pallas_call(
        _layer2_body,
        grid=grid,
        in_specs=[
            pl.BlockSpec((NC, BM, 128), lambda i: (0, i, 0)),
            pl.BlockSpec((NC, BM, 128), lambda i: (0, i, 0)),
            pl.BlockSpec((BM, 128), lambda i: (i, 0)),
            pl.BlockSpec((BM, 128), lambda i: (i, 0)),
            pl.BlockSpec((BM, 1), lambda i: (i, 0)),
            pl.BlockSpec((1, F_HID), lambda i: (0, 0)),
            pl.BlockSpec((F_HID, F2P), lambda i: (0, 0)),
        ],
        out_specs=pl.BlockSpec((BM, F2P), lambda i: (i, 0)),
        out_shape=jax.ShapeDtypeStruct((NP_, F2P), jnp.float32),
    )(agga, aggb, h1sa, h1sb, dis, b1r, W2p)

    agg2 = agg128_k(h2s, srcp, dstp).reshape(NC, NP_, F2P)

    BO = 1000
    out = pl.pallas_call(
        _out_body,
        grid=(N // BO,),
        in_specs=[
            pl.BlockSpec((NC, BO, F2P), lambda i: (0, i, 0)),
            pl.BlockSpec((BO, F2P), lambda i: (i, 0)),
            pl.BlockSpec((BO, 1), lambda i: (i, 0)),
            pl.BlockSpec((1, F2P), lambda i: (0, 0)),
        ],
        out_specs=pl.BlockSpec((BO, F_OUT), lambda i: (i, 0)),
        out_shape=jax.ShapeDtypeStruct((N, F_OUT), jnp.float32),
    )(agg2, h2s, dis, b2r)

    return out
